# Initial kernel scaffold; baseline (speedup 1.0000x reference)
import jax
import jax.numpy as jnp
from jax.experimental import pallas as pl


def kernel(pivot_words, target_words, doc_vectors, W, noise):
    raise NotImplementedError("write your pallas kernel here")


# speedup vs baseline: 3.6023x; 3.6023x over previous
import functools

import jax
import jax.numpy as jnp
from jax import lax
from jax.experimental import pallas as pl
from jax.experimental.pallas import tpu as pltpu
from jax.experimental.pallas import tpu_sc as plsc

VOCAB_N = 100000
D = 128
BATCH = 4096
WIN_N = 20
NEG_N = 10
NPAIR = WIN_N + WIN_N * NEG_N
NP_PAD = 224
HALF = NP_PAD // 2
EPS = 1e-08

NCORE = 2
NSUB = 16
LANES = 16
NWORK = NCORE * NSUB
BPW = BATCH // NWORK
DV = D // LANES


def _take16(x, idx):
    return lax.gather(
        x, idx[:, None],
        dimension_numbers=lax.GatherDimensionNumbers(
            offset_dims=(), collapsed_slice_dims=(0,), start_index_map=(0,)),
        slice_sizes=(1,), mode=lax.GatherScatterMode.PROMISE_IN_BOUNDS)


def _sc_body(w_hbm, piv_hbm, doc_hbm, idx_hbm, lg_hbm,
             idx_v, ctx_v, rows_v, lg_v, pividx_v, gsem, osem):
    wid = lax.axis_index("s") * NCORE + lax.axis_index("c")
    base = wid * BPW

    pltpu.sync_copy(piv_hbm.at[pl.ds(base, BPW)], pividx_v)
    pltpu.async_copy(w_hbm.at[pividx_v], rows_v.at[0, pl.ds(0, BPW)], gsem).wait()
    pltpu.sync_copy(doc_hbm.at[pl.ds(base, BPW)], ctx_v)

    def add_row(r, _):
        for j in range(DV):
            sl = pl.ds(j * LANES, LANES)
            ctx_v[r, sl] = ctx_v[r, sl] + rows_v[0, r, sl]
        return ()
    lax.fori_loop(0, BPW, add_row, (), unroll=2)

    pltpu.sync_copy(idx_hbm.at[pl.ds(base, BPW)], idx_v)

    lanes = lax.iota(jnp.int32, LANES)
    perms = [lanes ^ (1 << k) for k in range(4)]
    lmask = [lanes == j for j in range(LANES)]

    def do_b(i, _):
        c1 = pltpu.make_async_copy(
            w_hbm.at[idx_v.at[i, 0]], rows_v.at[0, pl.ds(0, HALF)], gsem)
        c2 = pltpu.make_async_copy(
            w_hbm.at[idx_v.at[i, 1]], rows_v.at[0, pl.ds(HALF, HALF)], gsem)
        c1.start()
        c2.start()
        c1.wait()
        c2.wait()

        cvec = [ctx_v[i, pl.ds(j * LANES, LANES)] for j in range(DV)]

        def do_g(g, _):
            out = jnp.zeros((LANES,), jnp.float32)
            for q in range(LANES):
                p = g * LANES + q
                acc = rows_v[0, p, pl.ds(0, LANES)] * cvec[0]
                for j in range(1, DV):
                    acc = acc + rows_v[0, p, pl.ds(j * LANES, LANES)] * cvec[j]
                for pm in perms:
                    acc = acc + _take16(acc, pm)
                out = jnp.where(lmask[q], acc, out)
            lg_v[pl.ds(g * LANES, LANES)] = out
            return ()
        lax.fori_loop(0, NP_PAD // LANES, do_g, ())

        pltpu.sync_copy(lg_v, lg_hbm.at[base + i])
        return ()
    lax.fori_loop(0, BPW, do_b, ())


_sc_logits = functools.partial(
    pl.kernel,
    out_type=jax.ShapeDtypeStruct((BATCH, NP_PAD), jnp.float32),
    mesh=plsc.VectorSubcoreMesh(
        core_axis_name="c", subcore_axis_name="s",
        num_cores=NCORE, num_subcores=NSUB),
    scratch_types=[
        pltpu.VMEM((BPW, 2, HALF), jnp.int32),
        pltpu.VMEM((BPW, D), jnp.float32),
        pltpu.VMEM((2, NP_PAD, D), jnp.float32),
        pltpu.VMEM((NP_PAD,), jnp.float32),
        pltpu.VMEM((BPW,), jnp.int32),
        pltpu.SemaphoreType.DMA,
        pltpu.SemaphoreType.DMA,
    ],
)(_sc_body)


def _tc_body(lg_ref, out_ref):
    i = pl.program_id(0)
    x = lg_ref[...]
    col = lax.broadcasted_iota(jnp.int32, x.shape, 1)
    lp = jnp.where(col < WIN_N, x, -x)
    y = jnp.log(jnp.clip(jax.nn.sigmoid(lp), EPS))
    y = jnp.where(col < NPAIR, y, 0.0)
    s = jnp.sum(y)

    @pl.when(i == 0)
    def _():
        out_ref[0, 0] = 0.0
    out_ref[0, 0] += s


_TC_ROWS = 256

_tc_reduce = pl.pallas_call(
    _tc_body,
    grid=(BATCH // _TC_ROWS,),
    in_specs=[pl.BlockSpec((_TC_ROWS, NP_PAD), lambda i: (i, 0))],
    out_specs=pl.BlockSpec(
        block_shape=(1, 1), index_map=lambda i: (0, 0),
        memory_space=pltpu.SMEM),
    out_shape=jax.ShapeDtypeStruct((1, 1), jnp.float32),
)


def kernel(pivot_words, target_words, doc_vectors, W, noise):
    piv = pivot_words.astype(jnp.int32)
    idx = jnp.concatenate(
        [target_words.astype(jnp.int32), noise.astype(jnp.int32),
         jnp.zeros((BATCH, NP_PAD - NPAIR), jnp.int32)],
        axis=1).reshape(BATCH, 2, HALF)
    lg = _sc_logits(W, piv, doc_vectors, idx)
    total = _tc_reduce(lg)
    return -(total[0, 0] / BATCH)
